# baseline (speedup 1.0000x reference)
import functools

import jax
import jax.numpy as jnp
from jax import lax
from jax.experimental import pallas as pl
from jax.experimental.pallas import tpu as pltpu
from jax.experimental.pallas import tpu_sc as plsc

_NUM_CORES = 2
_NUM_SUBCORES = 16
_NUM_WORKERS = _NUM_CORES * _NUM_SUBCORES

_CHUNK = 512
_LANES = 128
_SUB = 8


def _make_lookup(batch: int, seq: int, vocab: int, dim: int):
    total = batch * seq
    per_worker = total // _NUM_WORKERS
    num_chunks = per_worker // _CHUNK
    chunks_per_s = batch // _CHUNK
    tj_n = dim // _SUB
    tv_per_chunk = _CHUNK // _LANES
    slab = tv_per_chunk * _SUB * _LANES
    minor = (batch // _LANES) * _SUB * _LANES
    assert num_chunks % 2 == 0 and dim == 32 and _CHUNK % _LANES == 0

    mesh = plsc.VectorSubcoreMesh(core_axis_name="c", subcore_axis_name="s")

    @functools.partial(
        pl.kernel,
        mesh=mesh,
        out_type=jax.ShapeDtypeStruct((seq, tj_n, minor), jnp.float32),
        scratch_types=[
            pltpu.VMEM((per_worker,), jnp.int32),
            [pltpu.VMEM((_CHUNK, dim), jnp.float32) for _ in range(2)],
            [pltpu.VMEM((tj_n * slab,), jnp.float32) for _ in range(2)],
            [pltpu.SemaphoreType.DMA for _ in range(2)],
            [pltpu.SemaphoreType.DMA for _ in range(2)],
        ],
        compiler_params=pltpu.CompilerParams(
            use_tc_tiling_on_sc=False, needs_layout_passes=False),
    )
    def lookup(idx_hbm, table_hbm, out_hbm, idx_v, rows, trans, gsems, osems):
        wid = lax.axis_index("s") * _NUM_CORES + lax.axis_index("c")
        qbase = wid * num_chunks
        pltpu.sync_copy(idx_hbm.at[pl.ds(wid * per_worker, per_worker)], idx_v)

        lane = lax.iota(jnp.int32, 16)
        dst_pat = []
        for c in range(dim // 16):
            j = lane + (c * 16)
            dst_pat.append((j >> 3) * slab + (j & 7) * _LANES)

        def start_gather(i, p):
            pltpu.async_copy(
                table_hbm.at[idx_v.at[pl.ds(i * _CHUNK, _CHUNK)]],
                rows[p], gsems[p])

        def wait_gather(i, p):
            pltpu.make_async_copy(
                table_hbm.at[idx_v.at[pl.ds(i * _CHUNK, _CHUNK)]],
                rows[p], gsems[p]).wait()

        def out_copies(i, p):
            q = qbase + i
            s = q // chunks_per_s
            tv0 = (q % chunks_per_s) * tv_per_chunk
            return [
                pltpu.make_async_copy(
                    trans[p].at[pl.ds(tj * slab, slab)],
                    out_hbm.at[s, tj, pl.ds(tv0 * _LANES * _SUB, slab)],
                    osems[p])
                for tj in range(tj_n)
            ]

        def transpose_chunk(p):
            rows_p, trans_p = rows[p], trans[p]

            def body(m, carry):
                for k in range(16):
                    t = m * 16 + k
                    base = (t >> 7) * (_SUB * _LANES) + (t & (_LANES - 1))
                    for c in range(dim // 16):
                        v = rows_p[t, pl.ds(c * 16, 16)]
                        plsc.store_scatter(trans_p, [dst_pat[c] + base], v)
                return carry

            lax.fori_loop(0, _CHUNK // 16, body, 0)

        def step(i, p):
            wait_gather(i, p)

            @pl.when(i >= 2)
            def _():
                for cp in out_copies(i - 2, p):
                    cp.wait()

            transpose_chunk(p)
            for cp in out_copies(i, p):
                cp.start()

            @pl.when(i + 2 < num_chunks)
            def _():
                start_gather(i + 2, p)

        start_gather(0, 0)
        start_gather(1, 1)

        def outer(t, carry):
            step(2 * t, 0)
            step(2 * t + 1, 1)
            return carry

        lax.fori_loop(0, num_chunks // 2, outer, 0)

        for cp in out_copies(num_chunks - 2, 0):
            cp.wait()
        for cp in out_copies(num_chunks - 1, 1):
            cp.wait()

    return lookup


def kernel(token_ids, weights):
    batch, seq = token_ids.shape
    vocab, dim = weights.shape
    flat_idx = token_ids.T.reshape(batch * seq).astype(jnp.int32)
    lookup = _make_lookup(batch, seq, vocab, dim)
    out = lookup(flat_idx, weights)
    out5d = out.reshape(seq, dim // _SUB, batch // _LANES, _SUB, _LANES)
    return out5d.transpose(2, 4, 0, 1, 3).reshape(batch, seq, dim)


# speedup vs baseline: 1.0578x; 1.0578x over previous
import functools

import jax
import jax.numpy as jnp
from jax import lax
from jax.experimental import pallas as pl
from jax.experimental.pallas import tpu as pltpu
from jax.experimental.pallas import tpu_sc as plsc

_NUM_CORES = 2
_NUM_SUBCORES = 16
_NUM_WORKERS = _NUM_CORES * _NUM_SUBCORES

_CHUNK = 512
_LANES = 128
_SUB = 8


def _make_lookup(batch: int, seq: int, vocab: int, dim: int):
    total = batch * seq
    per_worker = total // _NUM_WORKERS
    num_chunks = per_worker // _CHUNK
    chunks_per_s = batch // _CHUNK
    tj_n = dim // _SUB
    tv_per_chunk = _CHUNK // _LANES
    slab = tv_per_chunk * _SUB * _LANES
    minor = (batch // _LANES) * _SUB * _LANES
    assert num_chunks % 2 == 0 and dim == 32 and _CHUNK % _LANES == 0

    mesh = plsc.VectorSubcoreMesh(core_axis_name="c", subcore_axis_name="s")

    @functools.partial(
        pl.kernel,
        mesh=mesh,
        out_type=jax.ShapeDtypeStruct((seq, tj_n, minor), jnp.float32),
        scratch_types=[
            pltpu.VMEM((per_worker,), jnp.int32),
            [pltpu.VMEM((_CHUNK, dim), jnp.float32) for _ in range(2)],
            [pltpu.VMEM((tj_n * slab,), jnp.float32) for _ in range(2)],
            [pltpu.SemaphoreType.DMA for _ in range(2)],
            [pltpu.SemaphoreType.DMA for _ in range(2)],
        ],
        compiler_params=pltpu.CompilerParams(
            use_tc_tiling_on_sc=False, needs_layout_passes=False),
    )
    def lookup(idx_hbm, table_hbm, out_hbm, idx_v, rows, trans, gsems, osems):
        wid = lax.axis_index("s") * _NUM_CORES + lax.axis_index("c")
        qbase = wid * num_chunks
        pltpu.sync_copy(idx_hbm.at[pl.ds(wid * per_worker, per_worker)], idx_v)

        lane = lax.iota(jnp.int32, 16)
        dst_pat = []
        for c in range(dim // 16):
            j = lane + (c * 16)
            dst_pat.append((j >> 3) * slab + (j & 7) * _LANES)

        def start_gather(i, p):
            pltpu.async_copy(
                table_hbm.at[idx_v.at[pl.ds(i * _CHUNK, _CHUNK)]],
                rows[p], gsems[p])

        def wait_gather(i, p):
            pltpu.make_async_copy(
                table_hbm.at[idx_v.at[pl.ds(i * _CHUNK, _CHUNK)]],
                rows[p], gsems[p]).wait()

        def out_copies(i, p):
            q = qbase + i
            s = q // chunks_per_s
            tv0 = (q % chunks_per_s) * tv_per_chunk
            return [
                pltpu.make_async_copy(
                    trans[p].at[pl.ds(tj * slab, slab)],
                    out_hbm.at[s, tj, pl.ds(tv0 * _LANES * _SUB, slab)],
                    osems[p])
                for tj in range(tj_n)
            ]

        def transpose_chunk(p):
            rows_p, trans_p = rows[p], trans[p]
            n_c = dim // 16

            def body(m, carry):
                b0 = (m >> 3) * (_SUB * _LANES) + (m & 7) * 16
                dst_m = [dst_pat[c] + b0 for c in range(n_c)]
                vals = []
                for k in range(16):
                    t = m * 16 + k
                    for c in range(n_c):
                        vals.append(rows_p[t, pl.ds(c * 16, 16)])
                for k in range(16):
                    for c in range(n_c):
                        plsc.store_scatter(
                            trans_p, [dst_m[c] + k], vals[k * n_c + c])
                return carry

            lax.fori_loop(0, _CHUNK // 16, body, 0)

        def step(i, p):
            wait_gather(i, p)

            @pl.when(i >= 2)
            def _():
                for cp in out_copies(i - 2, p):
                    cp.wait()

            transpose_chunk(p)
            for cp in out_copies(i, p):
                cp.start()

            @pl.when(i + 2 < num_chunks)
            def _():
                start_gather(i + 2, p)

        start_gather(0, 0)
        start_gather(1, 1)

        def outer(t, carry):
            step(2 * t, 0)
            step(2 * t + 1, 1)
            return carry

        lax.fori_loop(0, num_chunks // 2, outer, 0)

        for cp in out_copies(num_chunks - 2, 0):
            cp.wait()
        for cp in out_copies(num_chunks - 1, 1):
            cp.wait()

    return lookup


def kernel(token_ids, weights):
    batch, seq = token_ids.shape
    vocab, dim = weights.shape
    flat_idx = token_ids.T.reshape(batch * seq).astype(jnp.int32)
    lookup = _make_lookup(batch, seq, vocab, dim)
    out = lookup(flat_idx, weights)
    out5d = out.reshape(seq, dim // _SUB, batch // _LANES, _SUB, _LANES)
    return out5d.transpose(2, 4, 0, 1, 3).reshape(batch, seq, dim)
